# Initial kernel scaffold; baseline (speedup 1.0000x reference)
import jax
import jax.numpy as jnp
from jax.experimental import pallas as pl


def kernel(x, edge_index, edge_type, W1, root1, b1, W2, root2, b2, Wc, bc):
    raise NotImplementedError("write your pallas kernel here")


# speedup vs baseline: 10.7824x; 10.7824x over previous
import functools

import jax
import jax.numpy as jnp
from jax import lax
from jax.experimental import pallas as pl
from jax.experimental.pallas import tpu as pltpu
from jax.experimental.pallas import tpu_sc as plsc

N = 10000
E = 320000
R = 8
D = 128
NCORES = 2
NSUB = 16
NW = NCORES * NSUB

K = 128
EPT = 10112
EPAD = NW * EPT
HTAB = 81920
HPT = HTAB // NSUB
SROWS = 10240


def _mesh():
    return plsc.VectorSubcoreMesh(core_axis_name="c", subcore_axis_name="s",
                                  num_cores=NCORES, num_subcores=NSUB)


def _wid():
    return lax.axis_index("c") * NSUB + lax.axis_index("s")


def _sc_counts_body(dst_hbm, et_hbm, out_hbm, dst_v, et_v, cidx_v, ones_v,
                    zero_v, hist_sh):
    core = lax.axis_index("c")
    sub = lax.axis_index("s")
    w = _wid()

    def _fill(i, _):
        sl = pl.ds(i * 16, 16)
        ones_v[sl] = jnp.ones((16,), jnp.int32)
        return 0

    lax.fori_loop(0, K // 16, _fill, 0)

    def _zfill(i, _):
        zero_v[pl.ds(i * 16, 16)] = jnp.zeros((16,), jnp.int32)
        return 0

    lax.fori_loop(0, 1024 // 16, _zfill, 0)

    for j in range(HPT // 1024):
        pltpu.sync_copy(zero_v, hist_sh.at[pl.ds(sub * HPT + j * 1024, 1024)])

    pltpu.sync_copy(dst_hbm.at[pl.ds(w * EPT, EPT)], dst_v)
    pltpu.sync_copy(et_hbm.at[pl.ds(w * EPT, EPT)], et_v)

    plsc.subcore_barrier()

    def _hist(i, _):
        base = i * K
        for s in range(K // 16):
            sl = pl.ds(base + s * 16, 16)
            cidx_v[pl.ds(s * 16, 16)] = dst_v[sl] * R + et_v[sl]
        pltpu.sync_copy(ones_v, hist_sh.at[cidx_v], add=True)
        return 0

    lax.fori_loop(0, EPT // K, _hist, 0)

    plsc.subcore_barrier()

    pltpu.sync_copy(hist_sh.at[pl.ds(sub * HPT, HPT)],
                    out_hbm.at[core, pl.ds(sub * HPT, HPT)])


@functools.cache
def _sc_counts():
    return pl.kernel(
        _sc_counts_body,
        out_type=jax.ShapeDtypeStruct((NCORES, HTAB), jnp.int32),
        mesh=_mesh(),
        scratch_types=[
            pltpu.VMEM((EPT,), jnp.int32),
            pltpu.VMEM((EPT,), jnp.int32),
            pltpu.VMEM((K,), jnp.int32),
            pltpu.VMEM((K,), jnp.int32),
            pltpu.VMEM((1024,), jnp.int32),
            pltpu.VMEM_SHARED((HTAB,), jnp.int32),
        ],
    )


def _sc_scatter_body(xw_hbm, winv_hbm, src_hbm, et_hbm, dst_hbm, out_hbm,
                     src_v, et_v, dst_v, g_v, d_v, cidx_v, wv, rows_v,
                     acc_sh):
    core = lax.axis_index("c")
    sub = lax.axis_index("s")
    w = _wid()

    zrow = jnp.zeros((16,), jnp.float32)

    def _zero_zb(i, _):
        for s in range(8):
            rows_v[i, pl.ds(s * 16, 16)] = zrow
        return 0

    lax.fori_loop(0, K, _zero_zb, 0)

    for j in range(5):
        pltpu.sync_copy(rows_v, acc_sh.at[pl.ds(sub * 640 + j * 128, 128)])

    pltpu.sync_copy(src_hbm.at[pl.ds(w * EPT, EPT)], src_v)
    pltpu.sync_copy(et_hbm.at[pl.ds(w * EPT, EPT)], et_v)
    pltpu.sync_copy(dst_hbm.at[pl.ds(w * EPT, EPT)], dst_v)

    plsc.subcore_barrier()

    def _chunk(i, _):
        base = i * K
        for s in range(K // 16):
            sl = pl.ds(base + s * 16, 16)
            osl = pl.ds(s * 16, 16)
            ev = et_v[sl]
            dv = dst_v[sl]
            g_v[osl] = ev * N + src_v[sl]
            d_v[osl] = dv
            cidx_v[osl] = dv * R + ev
        pltpu.sync_copy(xw_hbm.at[g_v], rows_v)
        pltpu.sync_copy(winv_hbm.at[cidx_v], wv)

        def _scale(s, _):
            wv16 = wv[pl.ds(s * 16, 16)]
            for j in range(16):
                wvec = jnp.full((16,), wv16[j], jnp.float32)
                row = s * 16 + j
                for c in range(D // 16):
                    csl = pl.ds(c * 16, 16)
                    rows_v[row, csl] = rows_v[row, csl] * wvec
            return 0

        lax.fori_loop(0, K // 16, _scale, 0)

        pltpu.sync_copy(rows_v, acc_sh.at[d_v], add=True)
        return 0

    lax.fori_loop(0, EPT // K, _chunk, 0)

    plsc.subcore_barrier()

    pltpu.sync_copy(acc_sh.at[pl.ds(sub * 640, 640)],
                    out_hbm.at[core, pl.ds(sub * 640, 640)])


@functools.cache
def _sc_scatter():
    return pl.kernel(
        _sc_scatter_body,
        out_type=jax.ShapeDtypeStruct((NCORES, SROWS, D), jnp.float32),
        mesh=_mesh(),
        scratch_types=[
            pltpu.VMEM((EPT,), jnp.int32),
            pltpu.VMEM((EPT,), jnp.int32),
            pltpu.VMEM((EPT,), jnp.int32),
            pltpu.VMEM((K,), jnp.int32),
            pltpu.VMEM((K,), jnp.int32),
            pltpu.VMEM((K,), jnp.int32),
            pltpu.VMEM((K,), jnp.float32),
            pltpu.VMEM((K, D), jnp.float32),
            pltpu.VMEM_SHARED((SROWS, D), jnp.float32),
        ],
    )


_BN = 1000


def _tc_xw_body(x_ref, w_ref, o_ref):
    o_ref[0] = jnp.dot(x_ref[...], w_ref[0],
                       preferred_element_type=jnp.float32)


def _tc_xw(x, W):
    return pl.pallas_call(
        _tc_xw_body,
        grid=(R, N // _BN),
        in_specs=[
            pl.BlockSpec((_BN, D), lambda r, n: (n, 0)),
            pl.BlockSpec((1, D, D), lambda r, n: (r, 0, 0)),
        ],
        out_specs=pl.BlockSpec((1, _BN, D), lambda r, n: (r, n, 0)),
        out_shape=jax.ShapeDtypeStruct((R, N, D), jnp.float32),
    )(x, W)


def _tc_combine1_body(p0_ref, p1_ref, x_ref, root_ref, b_ref, o_ref):
    h = (p0_ref[...] + p1_ref[...]
         + jnp.dot(x_ref[...], root_ref[...],
                   preferred_element_type=jnp.float32) + b_ref[...])
    o_ref[...] = jnp.maximum(h, 0.0)


def _tc_combine1(p0, p1, x, root, b):
    return pl.pallas_call(
        _tc_combine1_body,
        grid=(N // _BN,),
        in_specs=[
            pl.BlockSpec((_BN, D), lambda n: (n, 0)),
            pl.BlockSpec((_BN, D), lambda n: (n, 0)),
            pl.BlockSpec((_BN, D), lambda n: (n, 0)),
            pl.BlockSpec((D, D), lambda n: (0, 0)),
            pl.BlockSpec((1, D), lambda n: (0, 0)),
        ],
        out_specs=pl.BlockSpec((_BN, D), lambda n: (n, 0)),
        out_shape=jax.ShapeDtypeStruct((N, D), jnp.float32),
    )(p0, p1, x, root, b)


def _tc_combine2_body(p0_ref, p1_ref, x_ref, root_ref, b_ref, wc_ref,
                      bc_ref, h_ref, lg_ref):
    h = (p0_ref[...] + p1_ref[...]
         + jnp.dot(x_ref[...], root_ref[...],
                   preferred_element_type=jnp.float32) + b_ref[...])
    h_ref[...] = h
    lg_ref[...] = jnp.dot(h, wc_ref[...],
                          preferred_element_type=jnp.float32) + bc_ref[...]


def _tc_combine2(p0, p1, x, root, b, wc, bc):
    return pl.pallas_call(
        _tc_combine2_body,
        grid=(N // _BN,),
        in_specs=[
            pl.BlockSpec((_BN, D), lambda n: (n, 0)),
            pl.BlockSpec((_BN, D), lambda n: (n, 0)),
            pl.BlockSpec((_BN, D), lambda n: (n, 0)),
            pl.BlockSpec((D, D), lambda n: (0, 0)),
            pl.BlockSpec((1, D), lambda n: (0, 0)),
            pl.BlockSpec((D, D), lambda n: (0, 0)),
            pl.BlockSpec((1, D), lambda n: (0, 0)),
        ],
        out_specs=[
            pl.BlockSpec((_BN, D), lambda n: (n, 0)),
            pl.BlockSpec((_BN, D), lambda n: (n, 0)),
        ],
        out_shape=[
            jax.ShapeDtypeStruct((N, D), jnp.float32),
            jax.ShapeDtypeStruct((N, D), jnp.float32),
        ],
    )(p0, p1, x, root, b, wc, bc)


def kernel(x, edge_index, edge_type, W1, root1, b1, W2, root2, b2, Wc, bc):
    src = edge_index[0]
    dst = edge_index[1]

    pad = EPAD - E
    src_p = jnp.pad(src, (0, pad))
    et_p = jnp.pad(edge_type, (0, pad))
    dst_p = jnp.pad(dst, (0, pad), constant_values=N)

    counts = _sc_counts()(dst_p, et_p)
    cnt = (counts[0] + counts[1]).astype(jnp.float32)
    winv = 1.0 / jnp.maximum(cnt, 1.0)

    b1r = b1.reshape(1, D)
    b2r = b2.reshape(1, D)
    wc_pad = jnp.pad(Wc, ((0, 0), (0, D - Wc.shape[1])))
    bc_pad = jnp.pad(bc, (0, D - bc.shape[0])).reshape(1, D)

    xw1 = _tc_xw(x, W1).reshape(R * N, D)
    part1 = _sc_scatter()(xw1, winv, src_p, et_p, dst_p)
    h = _tc_combine1(part1[0], part1[1], x, root1, b1r)

    xw2 = _tc_xw(h, W2).reshape(R * N, D)
    part2 = _sc_scatter()(xw2, winv, src_p, et_p, dst_p)
    h2, logits_pad = _tc_combine2(part2[0], part2[1], h, root2, b2r,
                                  wc_pad, bc_pad)

    return (h2, logits_pad[:, :Wc.shape[1]])


# speedup vs baseline: 14.6123x; 1.3552x over previous
import functools

import jax
import jax.numpy as jnp
from jax import lax
from jax.experimental import pallas as pl
from jax.experimental.pallas import tpu as pltpu
from jax.experimental.pallas import tpu_sc as plsc

N = 10000
E = 320000
R = 8
D = 128
NCORES = 2
NSUB = 16
NW = NCORES * NSUB

K = 64
NCH = 158
EPT = NCH * K
EPAD = NW * EPT
HTAB = 81920
HPT = HTAB // NSUB
SROWS = 10240


def _mesh():
    return plsc.VectorSubcoreMesh(core_axis_name="c", subcore_axis_name="s",
                                  num_cores=NCORES, num_subcores=NSUB)


def _wid():
    return lax.axis_index("c") * NSUB + lax.axis_index("s")


def _sc_counts_body(dst_hbm, et_hbm, out_hbm, dst_v, et_v, cidx_v, ones_v,
                    zero_v, hist_sh):
    core = lax.axis_index("c")
    sub = lax.axis_index("s")
    w = _wid()

    def _fill(i, _):
        sl = pl.ds(i * 16, 16)
        ones_v[sl] = jnp.ones((16,), jnp.int32)
        return 0

    lax.fori_loop(0, K // 16, _fill, 0)

    def _zfill(i, _):
        zero_v[pl.ds(i * 16, 16)] = jnp.zeros((16,), jnp.int32)
        return 0

    lax.fori_loop(0, 1024 // 16, _zfill, 0)

    for j in range(HPT // 1024):
        pltpu.sync_copy(zero_v, hist_sh.at[pl.ds(sub * HPT + j * 1024, 1024)])

    pltpu.sync_copy(dst_hbm.at[pl.ds(w * EPT, EPT)], dst_v)
    pltpu.sync_copy(et_hbm.at[pl.ds(w * EPT, EPT)], et_v)

    plsc.subcore_barrier()

    def _hist(i, _):
        base = i * K
        for s in range(K // 16):
            sl = pl.ds(base + s * 16, 16)
            cidx_v[pl.ds(s * 16, 16)] = dst_v[sl] * R + et_v[sl]
        pltpu.sync_copy(ones_v, hist_sh.at[cidx_v], add=True)
        return 0

    lax.fori_loop(0, EPT // K, _hist, 0)

    plsc.subcore_barrier()

    pltpu.sync_copy(hist_sh.at[pl.ds(sub * HPT, HPT)],
                    out_hbm.at[core, pl.ds(sub * HPT, HPT)])


@functools.cache
def _sc_counts():
    return pl.kernel(
        _sc_counts_body,
        out_type=jax.ShapeDtypeStruct((NCORES, HTAB), jnp.int32),
        mesh=_mesh(),
        scratch_types=[
            pltpu.VMEM((EPT,), jnp.int32),
            pltpu.VMEM((EPT,), jnp.int32),
            pltpu.VMEM((K,), jnp.int32),
            pltpu.VMEM((K,), jnp.int32),
            pltpu.VMEM((1024,), jnp.int32),
            pltpu.VMEM_SHARED((HTAB,), jnp.int32),
        ],
    )


def _sc_scatter_body(xw_hbm, winv_hbm, src_hbm, et_hbm, dst_hbm, out_hbm,
                     src_v, et_v, dst_v, g0, d0, ci0, wv0, rows0,
                     g1, d1, ci1, wv1, rows1, acc_sh, sem0, sem1):
    core = lax.axis_index("c")
    sub = lax.axis_index("s")
    w = _wid()

    sets = ((g0, d0, ci0, wv0, rows0, sem0), (g1, d1, ci1, wv1, rows1, sem1))

    zrow = jnp.zeros((16,), jnp.float32)

    def _zero_zb(i, _):
        for s in range(8):
            rows0[i, pl.ds(s * 16, 16)] = zrow
        return 0

    lax.fori_loop(0, K, _zero_zb, 0)

    for j in range(640 // K):
        pltpu.sync_copy(rows0, acc_sh.at[pl.ds(sub * 640 + j * K, K)])

    pltpu.sync_copy(src_hbm.at[pl.ds(w * EPT, EPT)], src_v)
    pltpu.sync_copy(et_hbm.at[pl.ds(w * EPT, EPT)], et_v)
    pltpu.sync_copy(dst_hbm.at[pl.ds(w * EPT, EPT)], dst_v)

    def _build_and_fire(c, p):
        g, d, ci, wvb, rows, sem = sets[p]
        base = c * K
        for s in range(K // 16):
            sl = pl.ds(base + s * 16, 16)
            osl = pl.ds(s * 16, 16)
            ev = et_v[sl]
            dv = dst_v[sl]
            g[osl] = ev * N + src_v[sl]
            d[osl] = dv
            ci[osl] = dv * R + ev
        pltpu.async_copy(xw_hbm.at[g], rows, sem)
        pltpu.async_copy(winv_hbm.at[ci], wvb, sem)

    def _drain_scale_scatter(c, p):
        g, d, ci, wvb, rows, sem = sets[p]
        pltpu.make_async_copy(xw_hbm.at[g], rows, sem).wait()
        pltpu.make_async_copy(winv_hbm.at[ci], wvb, sem).wait()

        def _scale(s, _):
            wv16 = wvb[pl.ds(s * 16, 16)]
            for j in range(16):
                wvec = jnp.full((16,), wv16[j], jnp.float32)
                row = s * 16 + j
                for cc in range(D // 16):
                    csl = pl.ds(cc * 16, 16)
                    rows[row, csl] = rows[row, csl] * wvec
            return 0

        lax.fori_loop(0, K // 16, _scale, 0)

        pltpu.sync_copy(rows, acc_sh.at[d], add=True)

    plsc.subcore_barrier()

    _build_and_fire(0, 0)

    def _body(i2, _):
        c0 = i2 * 2
        _build_and_fire(c0 + 1, 1)
        _drain_scale_scatter(c0, 0)

        @pl.when(c0 + 2 < NCH)
        def _():
            _build_and_fire(c0 + 2, 0)

        _drain_scale_scatter(c0 + 1, 1)
        return 0

    lax.fori_loop(0, NCH // 2, _body, 0)

    plsc.subcore_barrier()

    pltpu.sync_copy(acc_sh.at[pl.ds(sub * 640, 640)],
                    out_hbm.at[core, pl.ds(sub * 640, 640)])


@functools.cache
def _sc_scatter():
    return pl.kernel(
        _sc_scatter_body,
        out_type=jax.ShapeDtypeStruct((NCORES, SROWS, D), jnp.float32),
        mesh=_mesh(),
        scratch_types=(
            [pltpu.VMEM((EPT,), jnp.int32) for _ in range(3)]
            + [pltpu.VMEM((K,), jnp.int32) for _ in range(3)]
            + [pltpu.VMEM((K,), jnp.float32)]
            + [pltpu.VMEM((K, D), jnp.float32)]
            + [pltpu.VMEM((K,), jnp.int32) for _ in range(3)]
            + [pltpu.VMEM((K,), jnp.float32)]
            + [pltpu.VMEM((K, D), jnp.float32)]
            + [pltpu.VMEM_SHARED((SROWS, D), jnp.float32)]
            + [pltpu.SemaphoreType.DMA for _ in range(2)]
        ),
    )


_BN = 1000


def _tc_xw_body(x_ref, w_ref, o_ref):
    o_ref[0] = jnp.dot(x_ref[...], w_ref[0],
                       preferred_element_type=jnp.float32)


def _tc_xw(x, W):
    return pl.pallas_call(
        _tc_xw_body,
        grid=(R, N // _BN),
        in_specs=[
            pl.BlockSpec((_BN, D), lambda r, n: (n, 0)),
            pl.BlockSpec((1, D, D), lambda r, n: (r, 0, 0)),
        ],
        out_specs=pl.BlockSpec((1, _BN, D), lambda r, n: (r, n, 0)),
        out_shape=jax.ShapeDtypeStruct((R, N, D), jnp.float32),
    )(x, W)


def _tc_combine1_body(p0_ref, p1_ref, x_ref, root_ref, b_ref, o_ref):
    h = (p0_ref[...] + p1_ref[...]
         + jnp.dot(x_ref[...], root_ref[...],
                   preferred_element_type=jnp.float32) + b_ref[...])
    o_ref[...] = jnp.maximum(h, 0.0)


def _tc_combine1(p0, p1, x, root, b):
    return pl.pallas_call(
        _tc_combine1_body,
        grid=(N // _BN,),
        in_specs=[
            pl.BlockSpec((_BN, D), lambda n: (n, 0)),
            pl.BlockSpec((_BN, D), lambda n: (n, 0)),
            pl.BlockSpec((_BN, D), lambda n: (n, 0)),
            pl.BlockSpec((D, D), lambda n: (0, 0)),
            pl.BlockSpec((1, D), lambda n: (0, 0)),
        ],
        out_specs=pl.BlockSpec((_BN, D), lambda n: (n, 0)),
        out_shape=jax.ShapeDtypeStruct((N, D), jnp.float32),
    )(p0, p1, x, root, b)


def _tc_combine2_body(p0_ref, p1_ref, x_ref, root_ref, b_ref, wc_ref,
                      bc_ref, h_ref, lg_ref):
    h = (p0_ref[...] + p1_ref[...]
         + jnp.dot(x_ref[...], root_ref[...],
                   preferred_element_type=jnp.float32) + b_ref[...])
    h_ref[...] = h
    lg_ref[...] = jnp.dot(h, wc_ref[...],
                          preferred_element_type=jnp.float32) + bc_ref[...]


def _tc_combine2(p0, p1, x, root, b, wc, bc):
    return pl.pallas_call(
        _tc_combine2_body,
        grid=(N // _BN,),
        in_specs=[
            pl.BlockSpec((_BN, D), lambda n: (n, 0)),
            pl.BlockSpec((_BN, D), lambda n: (n, 0)),
            pl.BlockSpec((_BN, D), lambda n: (n, 0)),
            pl.BlockSpec((D, D), lambda n: (0, 0)),
            pl.BlockSpec((1, D), lambda n: (0, 0)),
            pl.BlockSpec((D, D), lambda n: (0, 0)),
            pl.BlockSpec((1, D), lambda n: (0, 0)),
        ],
        out_specs=[
            pl.BlockSpec((_BN, D), lambda n: (n, 0)),
            pl.BlockSpec((_BN, D), lambda n: (n, 0)),
        ],
        out_shape=[
            jax.ShapeDtypeStruct((N, D), jnp.float32),
            jax.ShapeDtypeStruct((N, D), jnp.float32),
        ],
    )(p0, p1, x, root, b, wc, bc)


def kernel(x, edge_index, edge_type, W1, root1, b1, W2, root2, b2, Wc, bc):
    src = edge_index[0]
    dst = edge_index[1]

    pad = EPAD - E
    src_p = jnp.pad(src, (0, pad))
    et_p = jnp.pad(edge_type, (0, pad))
    dst_p = jnp.pad(dst, (0, pad), constant_values=N)

    counts = _sc_counts()(dst_p, et_p)
    cnt = (counts[0] + counts[1]).astype(jnp.float32)
    winv = 1.0 / jnp.maximum(cnt, 1.0)

    b1r = b1.reshape(1, D)
    b2r = b2.reshape(1, D)
    wc_pad = jnp.pad(Wc, ((0, 0), (0, D - Wc.shape[1])))
    bc_pad = jnp.pad(bc, (0, D - bc.shape[0])).reshape(1, D)

    xw1 = _tc_xw(x, W1).reshape(R * N, D)
    part1 = _sc_scatter()(xw1, winv, src_p, et_p, dst_p)
    h = _tc_combine1(part1[0], part1[1], x, root1, b1r)

    xw2 = _tc_xw(h, W2).reshape(R * N, D)
    part2 = _sc_scatter()(xw2, winv, src_p, et_p, dst_p)
    h2, logits_pad = _tc_combine2(part2[0], part2[1], h, root2, b2r,
                                  wc_pad, bc_pad)

    return (h2, logits_pad[:, :Wc.shape[1]])


# speedup vs baseline: 14.9357x; 1.0221x over previous
import functools

import jax
import jax.numpy as jnp
from jax import lax
from jax.experimental import pallas as pl
from jax.experimental.pallas import tpu as pltpu
from jax.experimental.pallas import tpu_sc as plsc

N = 10000
E = 320000
R = 8
D = 128
NCORES = 2
NSUB = 16
NW = NCORES * NSUB

K = 64
EPTC = 10112
EPAD = NW * EPTC
NCH0 = 110
NCH1 = 206
EPT0 = NCH0 * K
EPT1 = NCH1 * K
EPTMAX = EPT1
HTAB = 81920
HPT = HTAB // NSUB
SROWS = 10240


def _mesh():
    return plsc.VectorSubcoreMesh(core_axis_name="c", subcore_axis_name="s",
                                  num_cores=NCORES, num_subcores=NSUB)


def _wid():
    return lax.axis_index("c") * NSUB + lax.axis_index("s")


def _sc_counts_body(cidx_hbm, out_hbm, cf_v, cidx_v, ones_v, zero_v, hist_sh):
    core = lax.axis_index("c")
    sub = lax.axis_index("s")
    w = _wid()

    def _fill(i, _):
        sl = pl.ds(i * 16, 16)
        ones_v[sl] = jnp.ones((16,), jnp.int32)
        return 0

    lax.fori_loop(0, K // 16, _fill, 0)

    def _zfill(i, _):
        zero_v[pl.ds(i * 16, 16)] = jnp.zeros((16,), jnp.int32)
        return 0

    lax.fori_loop(0, 1024 // 16, _zfill, 0)

    for j in range(HPT // 1024):
        pltpu.sync_copy(zero_v, hist_sh.at[pl.ds(sub * HPT + j * 1024, 1024)])

    pltpu.sync_copy(cidx_hbm.at[pl.ds(w * EPTC, EPTC)], cf_v)

    plsc.subcore_barrier()

    def _hist(i, _):
        base = i * K
        for s in range(K // 16):
            cidx_v[pl.ds(s * 16, 16)] = cf_v[pl.ds(base + s * 16, 16)]
        pltpu.sync_copy(ones_v, hist_sh.at[cidx_v], add=True)
        return 0

    lax.fori_loop(0, EPTC // K, _hist, 0)

    plsc.subcore_barrier()

    pltpu.sync_copy(hist_sh.at[pl.ds(sub * HPT, HPT)],
                    out_hbm.at[core, pl.ds(sub * HPT, HPT)])


@functools.cache
def _sc_counts():
    return pl.kernel(
        _sc_counts_body,
        out_type=jax.ShapeDtypeStruct((NCORES, HTAB), jnp.int32),
        mesh=_mesh(),
        scratch_types=[
            pltpu.VMEM((EPTC,), jnp.int32),
            pltpu.VMEM((K,), jnp.int32),
            pltpu.VMEM((K,), jnp.int32),
            pltpu.VMEM((1024,), jnp.int32),
            pltpu.VMEM_SHARED((HTAB,), jnp.int32),
        ],
    )


def _sc_scatter_body(xw_hbm, winv_hbm, gidx_hbm, cidx_hbm, out_hbm,
                     gf_v, cf_v, g0, d0, ci0, wv0, rows0,
                     g1, d1, ci1, wv1, rows1, acc_sh, sem0, sem1):
    core = lax.axis_index("c")
    sub = lax.axis_index("s")

    ebase = jnp.where(core == 0, sub * EPT0, 16 * EPT0 + sub * EPT1)
    nch = jnp.where(core == 0, NCH0, NCH1)

    sets = ((g0, d0, ci0, wv0, rows0, sem0), (g1, d1, ci1, wv1, rows1, sem1))

    zrow = jnp.zeros((16,), jnp.float32)

    def _zero_zb(i, _):
        for s in range(8):
            rows0[i, pl.ds(s * 16, 16)] = zrow
        return 0

    lax.fori_loop(0, K, _zero_zb, 0)

    for j in range(640 // K):
        pltpu.sync_copy(rows0, acc_sh.at[pl.ds(sub * 640 + j * K, K)])

    pltpu.sync_copy(gidx_hbm.at[pl.ds(ebase, EPTMAX)], gf_v)
    pltpu.sync_copy(cidx_hbm.at[pl.ds(ebase, EPTMAX)], cf_v)

    def _build_and_fire(c, p):
        g, d, ci, wvb, rows, sem = sets[p]
        base = c * K
        for s in range(K // 16):
            sl = pl.ds(base + s * 16, 16)
            osl = pl.ds(s * 16, 16)
            civ = cf_v[sl]
            g[osl] = gf_v[sl]
            ci[osl] = civ
            d[osl] = lax.shift_right_logical(civ, 3)
        pltpu.async_copy(xw_hbm.at[g], rows, sem)
        pltpu.async_copy(winv_hbm.at[ci], wvb, sem)

    def _drain_scale_scatter(c, p):
        g, d, ci, wvb, rows, sem = sets[p]
        pltpu.make_async_copy(xw_hbm.at[g], rows, sem).wait()
        pltpu.make_async_copy(winv_hbm.at[ci], wvb, sem).wait()

        def _scale(s, _):
            wv16 = wvb[pl.ds(s * 16, 16)]
            for j in range(16):
                wvec = jnp.full((16,), wv16[j], jnp.float32)
                row = s * 16 + j
                for cc in range(D // 16):
                    csl = pl.ds(cc * 16, 16)
                    rows[row, csl] = rows[row, csl] * wvec
            return 0

        lax.fori_loop(0, K // 16, _scale, 0)

        pltpu.sync_copy(rows, acc_sh.at[d], add=True)

    plsc.subcore_barrier()

    _build_and_fire(0, 0)

    def _body(i2, _):
        c0 = i2 * 2
        _build_and_fire(c0 + 1, 1)
        _drain_scale_scatter(c0, 0)

        @pl.when(c0 + 2 < nch)
        def _():
            _build_and_fire(c0 + 2, 0)

        _drain_scale_scatter(c0 + 1, 1)
        return 0

    lax.fori_loop(0, nch // 2, _body, 0)

    plsc.subcore_barrier()

    pltpu.sync_copy(acc_sh.at[pl.ds(sub * 640, 640)],
                    out_hbm.at[core, pl.ds(sub * 640, 640)])


@functools.cache
def _sc_scatter():
    return pl.kernel(
        _sc_scatter_body,
        out_type=jax.ShapeDtypeStruct((NCORES, SROWS, D), jnp.float32),
        mesh=_mesh(),
        scratch_types=(
            [pltpu.VMEM((EPTMAX,), jnp.int32) for _ in range(2)]
            + [pltpu.VMEM((K,), jnp.int32) for _ in range(3)]
            + [pltpu.VMEM((K,), jnp.float32)]
            + [pltpu.VMEM((K, D), jnp.float32)]
            + [pltpu.VMEM((K,), jnp.int32) for _ in range(3)]
            + [pltpu.VMEM((K,), jnp.float32)]
            + [pltpu.VMEM((K, D), jnp.float32)]
            + [pltpu.VMEM_SHARED((SROWS, D), jnp.float32)]
            + [pltpu.SemaphoreType.DMA for _ in range(2)]
        ),
    )


_BN = 1000


def _tc_xw_body(x_ref, w_ref, o_ref):
    o_ref[0] = jnp.dot(x_ref[...], w_ref[0],
                       preferred_element_type=jnp.float32)


def _tc_xw(x, W):
    return pl.pallas_call(
        _tc_xw_body,
        grid=(R, N // _BN),
        in_specs=[
            pl.BlockSpec((_BN, D), lambda r, n: (n, 0)),
            pl.BlockSpec((1, D, D), lambda r, n: (r, 0, 0)),
        ],
        out_specs=pl.BlockSpec((1, _BN, D), lambda r, n: (r, n, 0)),
        out_shape=jax.ShapeDtypeStruct((R, N, D), jnp.float32),
    )(x, W)


def _tc_combine1_body(p0_ref, p1_ref, x_ref, root_ref, b_ref, o_ref):
    h = (p0_ref[...] + p1_ref[...]
         + jnp.dot(x_ref[...], root_ref[...],
                   preferred_element_type=jnp.float32) + b_ref[...])
    o_ref[...] = jnp.maximum(h, 0.0)


def _tc_combine1(p0, p1, x, root, b):
    return pl.pallas_call(
        _tc_combine1_body,
        grid=(N // _BN,),
        in_specs=[
            pl.BlockSpec((_BN, D), lambda n: (n, 0)),
            pl.BlockSpec((_BN, D), lambda n: (n, 0)),
            pl.BlockSpec((_BN, D), lambda n: (n, 0)),
            pl.BlockSpec((D, D), lambda n: (0, 0)),
            pl.BlockSpec((1, D), lambda n: (0, 0)),
        ],
        out_specs=pl.BlockSpec((_BN, D), lambda n: (n, 0)),
        out_shape=jax.ShapeDtypeStruct((N, D), jnp.float32),
    )(p0, p1, x, root, b)


def _tc_combine2_body(p0_ref, p1_ref, x_ref, root_ref, b_ref, wc_ref,
                      bc_ref, h_ref, lg_ref):
    h = (p0_ref[...] + p1_ref[...]
         + jnp.dot(x_ref[...], root_ref[...],
                   preferred_element_type=jnp.float32) + b_ref[...])
    h_ref[...] = h
    lg_ref[...] = jnp.dot(h, wc_ref[...],
                          preferred_element_type=jnp.float32) + bc_ref[...]


def _tc_combine2(p0, p1, x, root, b, wc, bc):
    return pl.pallas_call(
        _tc_combine2_body,
        grid=(N // _BN,),
        in_specs=[
            pl.BlockSpec((_BN, D), lambda n: (n, 0)),
            pl.BlockSpec((_BN, D), lambda n: (n, 0)),
            pl.BlockSpec((_BN, D), lambda n: (n, 0)),
            pl.BlockSpec((D, D), lambda n: (0, 0)),
            pl.BlockSpec((1, D), lambda n: (0, 0)),
            pl.BlockSpec((D, D), lambda n: (0, 0)),
            pl.BlockSpec((1, D), lambda n: (0, 0)),
        ],
        out_specs=[
            pl.BlockSpec((_BN, D), lambda n: (n, 0)),
            pl.BlockSpec((_BN, D), lambda n: (n, 0)),
        ],
        out_shape=[
            jax.ShapeDtypeStruct((N, D), jnp.float32),
            jax.ShapeDtypeStruct((N, D), jnp.float32),
        ],
    )(p0, p1, x, root, b, wc, bc)


def kernel(x, edge_index, edge_type, W1, root1, b1, W2, root2, b2, Wc, bc):
    src = edge_index[0]
    dst = edge_index[1]

    pad = EPAD - E
    src_p = jnp.pad(src, (0, pad))
    et_p = jnp.pad(edge_type, (0, pad))
    dst_p = jnp.pad(dst, (0, pad), constant_values=N)
    gidx = et_p * N + src_p
    cidx = dst_p * R + et_p

    counts = _sc_counts()(cidx)
    cnt = (counts[0] + counts[1]).astype(jnp.float32)
    winv = 1.0 / jnp.maximum(cnt, 1.0)

    b1r = b1.reshape(1, D)
    b2r = b2.reshape(1, D)
    wc_pad = jnp.pad(Wc, ((0, 0), (0, D - Wc.shape[1])))
    bc_pad = jnp.pad(bc, (0, D - bc.shape[0])).reshape(1, D)

    xw1 = _tc_xw(x, W1).reshape(R * N, D)
    part1 = _sc_scatter()(xw1, winv, gidx, cidx)
    h = _tc_combine1(part1[0], part1[1], x, root1, b1r)

    xw2 = _tc_xw(h, W2).reshape(R * N, D)
    part2 = _sc_scatter()(xw2, winv, gidx, cidx)
    h2, logits_pad = _tc_combine2(part2[0], part2[1], h, root2, b2r,
                                  wc_pad, bc_pad)

    return (h2, logits_pad[:, :Wc.shape[1]])


# speedup vs baseline: 17.4280x; 1.1669x over previous
import functools

import jax
import jax.numpy as jnp
from jax import lax
from jax.experimental import pallas as pl
from jax.experimental.pallas import tpu as pltpu
from jax.experimental.pallas import tpu_sc as plsc

N = 10000
E = 320000
R = 8
D = 128
NCORES = 2
NSUB = 16
NW = NCORES * NSUB

K = 64
EPTC = 10112
EPAD = NW * EPTC
NCH0 = 206
NCH1 = 110
EPT0 = NCH0 * K
EPT1 = NCH1 * K
EPTMAX = EPT0
HTAB = 81920
HPT = HTAB // NSUB
SROWS = 10240


def _mesh():
    return plsc.VectorSubcoreMesh(core_axis_name="c", subcore_axis_name="s",
                                  num_cores=NCORES, num_subcores=NSUB)


def _wid():
    return lax.axis_index("c") * NSUB + lax.axis_index("s")


def _sc_counts_body(cidx_hbm, out_hbm, cf_v, cidx_v, ones_v, zero_v, hist_sh):
    core = lax.axis_index("c")
    sub = lax.axis_index("s")
    w = _wid()

    def _fill(i, _):
        sl = pl.ds(i * 16, 16)
        ones_v[sl] = jnp.ones((16,), jnp.int32)
        return 0

    lax.fori_loop(0, K // 16, _fill, 0)

    def _zfill(i, _):
        zero_v[pl.ds(i * 16, 16)] = jnp.zeros((16,), jnp.int32)
        return 0

    lax.fori_loop(0, 1024 // 16, _zfill, 0)

    for j in range(HPT // 1024):
        pltpu.sync_copy(zero_v, hist_sh.at[pl.ds(sub * HPT + j * 1024, 1024)])

    pltpu.sync_copy(cidx_hbm.at[pl.ds(w * EPTC, EPTC)], cf_v)

    plsc.subcore_barrier()

    def _hist(i, _):
        base = i * K
        for s in range(K // 16):
            cidx_v[pl.ds(s * 16, 16)] = cf_v[pl.ds(base + s * 16, 16)]
        pltpu.sync_copy(ones_v, hist_sh.at[cidx_v], add=True)
        return 0

    lax.fori_loop(0, EPTC // K, _hist, 0)

    plsc.subcore_barrier()

    pltpu.sync_copy(hist_sh.at[pl.ds(sub * HPT, HPT)],
                    out_hbm.at[core, pl.ds(sub * HPT, HPT)])


@functools.cache
def _sc_counts():
    return pl.kernel(
        _sc_counts_body,
        out_type=jax.ShapeDtypeStruct((NCORES, HTAB), jnp.int32),
        mesh=_mesh(),
        scratch_types=[
            pltpu.VMEM((EPTC,), jnp.int32),
            pltpu.VMEM((K,), jnp.int32),
            pltpu.VMEM((K,), jnp.int32),
            pltpu.VMEM((1024,), jnp.int32),
            pltpu.VMEM_SHARED((HTAB,), jnp.int32),
        ],
    )


def _sc_scatter_body(xw_hbm, winv_hbm, gidx_hbm, cidx_hbm, out_hbm,
                     gf_v, cf_v, g0, d0, ci0, wv0, rows0,
                     g1, d1, ci1, wv1, rows1, acc_sh, sem0, sem1):
    core = lax.axis_index("c")
    sub = lax.axis_index("s")

    ebase = jnp.where(core == 0, sub * EPT0, 16 * EPT0 + sub * EPT1)
    nch = jnp.where(core == 0, NCH0, NCH1)

    sets = ((g0, d0, ci0, wv0, rows0, sem0), (g1, d1, ci1, wv1, rows1, sem1))

    zrow = jnp.zeros((16,), jnp.float32)

    def _zero_zb(i, _):
        for s in range(8):
            rows0[i, pl.ds(s * 16, 16)] = zrow
        return 0

    lax.fori_loop(0, K, _zero_zb, 0)

    for j in range(640 // K):
        pltpu.sync_copy(rows0, acc_sh.at[pl.ds(sub * 640 + j * K, K)])

    pltpu.sync_copy(gidx_hbm.at[pl.ds(ebase, EPTMAX)], gf_v)
    pltpu.sync_copy(cidx_hbm.at[pl.ds(ebase, EPTMAX)], cf_v)

    def _build_and_fire(c, p):
        g, d, ci, wvb, rows, sem = sets[p]
        base = c * K
        for s in range(K // 16):
            sl = pl.ds(base + s * 16, 16)
            osl = pl.ds(s * 16, 16)
            civ = cf_v[sl]
            g[osl] = gf_v[sl]
            ci[osl] = civ
            d[osl] = lax.shift_right_logical(civ, 3)
        pltpu.async_copy(xw_hbm.at[g], rows, sem)
        pltpu.async_copy(winv_hbm.at[ci], wvb, sem)

    def _drain_scale_scatter(c, p):
        g, d, ci, wvb, rows, sem = sets[p]
        pltpu.make_async_copy(xw_hbm.at[g], rows, sem).wait()
        pltpu.make_async_copy(winv_hbm.at[ci], wvb, sem).wait()

        def _scale(s, _):
            wv16 = wvb[pl.ds(s * 16, 16)]
            for j in range(16):
                wvec = jnp.full((16,), wv16[j], jnp.float32)
                row = s * 16 + j
                for cc in range(D // 16):
                    csl = pl.ds(cc * 16, 16)
                    rows[row, csl] = rows[row, csl] * wvec
            return 0

        lax.fori_loop(0, K // 16, _scale, 0)

        pltpu.sync_copy(rows, acc_sh.at[d], add=True)

    plsc.subcore_barrier()

    _build_and_fire(0, 0)

    def _body(i2, _):
        c0 = i2 * 2
        _build_and_fire(c0 + 1, 1)
        _drain_scale_scatter(c0, 0)

        @pl.when(c0 + 2 < nch)
        def _():
            _build_and_fire(c0 + 2, 0)

        _drain_scale_scatter(c0 + 1, 1)
        return 0

    lax.fori_loop(0, nch // 2, _body, 0)

    plsc.subcore_barrier()

    pltpu.sync_copy(acc_sh.at[pl.ds(sub * 640, 640)],
                    out_hbm.at[core, pl.ds(sub * 640, 640)])


@functools.cache
def _sc_scatter():
    return pl.kernel(
        _sc_scatter_body,
        out_type=jax.ShapeDtypeStruct((NCORES, SROWS, D), jnp.float32),
        mesh=_mesh(),
        scratch_types=(
            [pltpu.VMEM((EPTMAX,), jnp.int32) for _ in range(2)]
            + [pltpu.VMEM((K,), jnp.int32) for _ in range(3)]
            + [pltpu.VMEM((K,), jnp.float32)]
            + [pltpu.VMEM((K, D), jnp.float32)]
            + [pltpu.VMEM((K,), jnp.int32) for _ in range(3)]
            + [pltpu.VMEM((K,), jnp.float32)]
            + [pltpu.VMEM((K, D), jnp.float32)]
            + [pltpu.VMEM_SHARED((SROWS, D), jnp.float32)]
            + [pltpu.SemaphoreType.DMA for _ in range(2)]
        ),
    )


_BN = 1000


def _tc_xw_body(x_ref, w_ref, o_ref):
    o_ref[0] = jnp.dot(x_ref[...], w_ref[0],
                       preferred_element_type=jnp.float32)


def _tc_xw(x, W):
    return pl.pallas_call(
        _tc_xw_body,
        grid=(R, N // _BN),
        in_specs=[
            pl.BlockSpec((_BN, D), lambda r, n: (n, 0)),
            pl.BlockSpec((1, D, D), lambda r, n: (r, 0, 0)),
        ],
        out_specs=pl.BlockSpec((1, _BN, D), lambda r, n: (r, n, 0)),
        out_shape=jax.ShapeDtypeStruct((R, N, D), jnp.float32),
    )(x, W)


def _tc_combine1_body(p0_ref, p1_ref, x_ref, root_ref, b_ref, o_ref):
    h = (p0_ref[...] + p1_ref[...]
         + jnp.dot(x_ref[...], root_ref[...],
                   preferred_element_type=jnp.float32) + b_ref[...])
    o_ref[...] = jnp.maximum(h, 0.0)


def _tc_combine1(p0, p1, x, root, b):
    return pl.pallas_call(
        _tc_combine1_body,
        grid=(N // _BN,),
        in_specs=[
            pl.BlockSpec((_BN, D), lambda n: (n, 0)),
            pl.BlockSpec((_BN, D), lambda n: (n, 0)),
            pl.BlockSpec((_BN, D), lambda n: (n, 0)),
            pl.BlockSpec((D, D), lambda n: (0, 0)),
            pl.BlockSpec((1, D), lambda n: (0, 0)),
        ],
        out_specs=pl.BlockSpec((_BN, D), lambda n: (n, 0)),
        out_shape=jax.ShapeDtypeStruct((N, D), jnp.float32),
    )(p0, p1, x, root, b)


def _tc_combine2_body(p0_ref, p1_ref, x_ref, root_ref, b_ref, wc_ref,
                      bc_ref, h_ref, lg_ref):
    h = (p0_ref[...] + p1_ref[...]
         + jnp.dot(x_ref[...], root_ref[...],
                   preferred_element_type=jnp.float32) + b_ref[...])
    h_ref[...] = h
    lg_ref[...] = jnp.dot(h, wc_ref[...],
                          preferred_element_type=jnp.float32) + bc_ref[...]


def _tc_combine2(p0, p1, x, root, b, wc, bc):
    return pl.pallas_call(
        _tc_combine2_body,
        grid=(N // _BN,),
        in_specs=[
            pl.BlockSpec((_BN, D), lambda n: (n, 0)),
            pl.BlockSpec((_BN, D), lambda n: (n, 0)),
            pl.BlockSpec((_BN, D), lambda n: (n, 0)),
            pl.BlockSpec((D, D), lambda n: (0, 0)),
            pl.BlockSpec((1, D), lambda n: (0, 0)),
            pl.BlockSpec((D, D), lambda n: (0, 0)),
            pl.BlockSpec((1, D), lambda n: (0, 0)),
        ],
        out_specs=[
            pl.BlockSpec((_BN, D), lambda n: (n, 0)),
            pl.BlockSpec((_BN, D), lambda n: (n, 0)),
        ],
        out_shape=[
            jax.ShapeDtypeStruct((N, D), jnp.float32),
            jax.ShapeDtypeStruct((N, D), jnp.float32),
        ],
    )(p0, p1, x, root, b, wc, bc)


def kernel(x, edge_index, edge_type, W1, root1, b1, W2, root2, b2, Wc, bc):
    src = edge_index[0]
    dst = edge_index[1]

    pad = EPAD - E
    src_p = jnp.pad(src, (0, pad))
    et_p = jnp.pad(edge_type, (0, pad))
    dst_p = jnp.pad(dst, (0, pad), constant_values=N)
    gidx = et_p * N + src_p
    cidx = dst_p * R + et_p

    counts = _sc_counts()(cidx)
    cnt = (counts[0] + counts[1]).astype(jnp.float32)
    winv = 1.0 / jnp.maximum(cnt, 1.0)

    b1r = b1.reshape(1, D)
    b2r = b2.reshape(1, D)
    wc_pad = jnp.pad(Wc, ((0, 0), (0, D - Wc.shape[1])))
    bc_pad = jnp.pad(bc, (0, D - bc.shape[0])).reshape(1, D)

    xw1 = _tc_xw(x, W1).reshape(R * N, D)
    part1 = _sc_scatter()(xw1, winv, gidx, cidx)
    h = _tc_combine1(part1[0], part1[1], x, root1, b1r)

    xw2 = _tc_xw(h, W2).reshape(R * N, D)
    part2 = _sc_scatter()(xw2, winv, gidx, cidx)
    h2, logits_pad = _tc_combine2(part2[0], part2[1], h, root2, b2r,
                                  wc_pad, bc_pad)

    return (h2, logits_pad[:, :Wc.shape[1]])
